# baseline (speedup 1.0000x reference)
import functools

import jax
import jax.numpy as jnp
from jax.experimental import pallas as pl
from jax.experimental.pallas import tpu as pltpu

E = 8
TOPK = 2
D = 1024
L = 8
B = 4096
LD = L * D
EPS = 1e-09

BT = 1024
KT = 4096
NB = B // BT
NK = LD // KT


def _compute_gates(logits, maskf):
    m = jnp.max(logits, axis=1, keepdims=True)
    ex = jnp.exp(logits - m)
    g = ex / jnp.sum(ex, axis=1, keepdims=True)
    g = g * maskf
    lane = jax.lax.broadcasted_iota(jnp.int32, g.shape, 1)
    m1 = jnp.max(g, axis=1, keepdims=True)
    i1 = jnp.min(jnp.where(g == m1, lane, E), axis=1, keepdims=True)
    sel1 = lane == i1
    g2 = jnp.where(sel1, -jnp.inf, g)
    m2 = jnp.max(g2, axis=1, keepdims=True)
    i2 = jnp.min(jnp.where(g2 == m2, lane, E), axis=1, keepdims=True)
    sel = sel1 | (lane == i2)
    g = jnp.where(sel, g, 0.0)
    return g / (jnp.sum(g, axis=1, keepdims=True) + EPS)


def _moe_body(logits_ref, mask_ref, x_ref, w_ref, b_ref, o_ref,
              acc_ref, gates_ref):
    e = pl.program_id(1)
    kt = pl.program_id(2)

    @pl.when((e == 0) & (kt == 0))
    def _init():
        gates = _compute_gates(logits_ref[...], mask_ref[...])
        gates_ref[...] = gates
        acc_ref[...] = jax.lax.dot_general(
            gates, b_ref[...], (((1,), (0,)), ((), ())),
            preferred_element_type=jnp.float32)

    partial = jax.lax.dot_general(
        x_ref[...], w_ref[0], (((1,), (1,)), ((), ())),
        preferred_element_type=jnp.float32)
    lane = jax.lax.broadcasted_iota(jnp.int32, (BT, E), 1)
    gcol = jnp.sum(jnp.where(lane == e, gates_ref[...], 0.0), axis=1,
                   keepdims=True)
    acc_ref[...] += gcol * partial

    @pl.when((e == E - 1) & (kt == NK - 1))
    def _done():
        o_ref[...] = acc_ref[...].astype(jnp.bfloat16)


@functools.partial(jax.jit, static_argnames=())
def kernel(cycle_curve_data, logits, moe_masks, W, b):
    x = cycle_curve_data.reshape(B, LD).astype(jnp.bfloat16)
    Wb = W.astype(jnp.bfloat16)
    maskf = (moe_masks == 1).astype(jnp.float32)

    out = pl.pallas_call(
        _moe_body,
        grid=(NB, E, NK),
        in_specs=[
            pl.BlockSpec((BT, E), lambda bt, e, kt: (bt, 0)),
            pl.BlockSpec((BT, E), lambda bt, e, kt: (bt, 0)),
            pl.BlockSpec((BT, KT), lambda bt, e, kt: (bt, kt)),
            pl.BlockSpec((1, D, KT), lambda bt, e, kt: (e, 0, kt)),
            pl.BlockSpec((E, D), lambda bt, e, kt: (0, 0)),
        ],
        out_specs=pl.BlockSpec((BT, D), lambda bt, e, kt: (bt, 0)),
        out_shape=jax.ShapeDtypeStruct((B, D), jnp.bfloat16),
        scratch_shapes=[
            pltpu.VMEM((BT, D), jnp.float32),
            pltpu.VMEM((BT, E), jnp.float32),
        ],
        compiler_params=pltpu.CompilerParams(
            dimension_semantics=("arbitrary", "arbitrary", "arbitrary"),
        ),
    )(logits, maskf, x, Wb, b)
    return out
